# baseline (speedup 1.0000x reference)
import jax
import jax.numpy as jnp
from jax.experimental import pallas as pl
from jax.experimental.pallas import tpu as pltpu


def _round_up(x, m):
    return (x + m - 1) // m * m


def _rnn_core(
    tok_ref, emb_ref, wih_ref, whh_ref, brnn_ref, wfc_ref, bfc_ref, out_ref,
    xa_ref, xb_ref, pa_ref, pb_ref,
):
    bp = out_ref.shape[0]
    hp = whh_ref.shape[0]
    seq_len = tok_ref.shape[0] // bp

    w_ih = wih_ref[...].astype(jnp.bfloat16)
    w_hh = whh_ref[...].astype(jnp.bfloat16)
    b_rnn = brnn_ref[...]

    bufs = (xa_ref, xb_ref)

    def gather_to(buf, t):
        base = t * bp
        for i in range(bp):
            idx = tok_ref[base + i]
            buf[pl.ds(i, 1), :] = emb_ref[pl.ds(idx, 1), :]

    pbufs = (pa_ref, pb_ref)

    gather_to(bufs[0], 0)
    gather_to(bufs[1], 1)
    pa_ref[...] = (
        jnp.dot(
            bufs[0][...].astype(jnp.bfloat16),
            w_ih,
            preferred_element_type=jnp.float32,
        )
        + b_rnn
    )

    half = bp // 2
    h_a = jnp.zeros((half, hp), jnp.bfloat16)
    h_b = jnp.zeros((half, hp), jnp.bfloat16)
    for t in range(seq_len):
        if t + 2 < seq_len:
            gather_to(bufs[t % 2], t + 2)
        if t + 1 < seq_len:
            pbufs[(t + 1) % 2][...] = (
                jnp.dot(
                    bufs[(t + 1) % 2][...].astype(jnp.bfloat16),
                    w_ih,
                    preferred_element_type=jnp.float32,
                )
                + b_rnn
            )
        p_cur = pbufs[t % 2]
        acc_a = p_cur[pl.ds(0, half), :] + jnp.dot(
            h_a, w_hh, preferred_element_type=jnp.float32
        )
        acc_b = p_cur[pl.ds(half, half), :] + jnp.dot(
            h_b, w_hh, preferred_element_type=jnp.float32
        )
        h_a = jnp.tanh(acc_a.astype(jnp.bfloat16))
        h_b = jnp.tanh(acc_b.astype(jnp.bfloat16))

    h = jnp.concatenate([h_a, h_b], axis=0)
    out_ref[...] = (
        jnp.dot(
            h.astype(jnp.float32),
            wfc_ref[...],
            preferred_element_type=jnp.float32,
        )
        + bfc_ref[...]
    )


def kernel(token_ids, emb_table, w_ih, w_hh, b_rnn, w_fc, b_fc):
    B, S = token_ids.shape
    E = emb_table.shape[1]
    H = w_ih.shape[1]
    V = w_fc.shape[1]

    Bp = _round_up(max(B, 8), 8)
    Hp = _round_up(H, 128)
    Vp = _round_up(V, 128)

    tok = jnp.pad(token_ids.T, ((0, 0), (0, Bp - B))).reshape(S * Bp)

    w_ih_p = jnp.pad(w_ih, ((0, 0), (0, Hp - H)))
    w_hh_p = jnp.pad(w_hh, ((0, Hp - H), (0, Hp - H)))
    b_rnn_p = jnp.pad(b_rnn, ((0, 0), (0, Hp - H)))
    w_fc_p = jnp.pad(w_fc, ((0, Hp - H), (0, Vp - V)))
    b_fc_p = jnp.pad(b_fc, ((0, 0), (0, Vp - V)))

    grid_spec = pltpu.PrefetchScalarGridSpec(
        num_scalar_prefetch=1,
        grid=(1,),
        in_specs=[
            pl.BlockSpec(emb_table.shape, lambda i, *_: (0, 0)),
            pl.BlockSpec((E, Hp), lambda i, *_: (0, 0)),
            pl.BlockSpec((Hp, Hp), lambda i, *_: (0, 0)),
            pl.BlockSpec((1, Hp), lambda i, *_: (0, 0)),
            pl.BlockSpec((Hp, Vp), lambda i, *_: (0, 0)),
            pl.BlockSpec((1, Vp), lambda i, *_: (0, 0)),
        ],
        out_specs=pl.BlockSpec((Bp, Vp), lambda i, *_: (0, 0)),
        scratch_shapes=[
            pltpu.VMEM((Bp, E), jnp.float32),
            pltpu.VMEM((Bp, E), jnp.float32),
            pltpu.VMEM((Bp, Hp), jnp.float32),
            pltpu.VMEM((Bp, Hp), jnp.float32),
        ],
    )

    out_padded = pl.pallas_call(
        _rnn_core,
        out_shape=jax.ShapeDtypeStruct((Bp, Vp), jnp.float32),
        grid_spec=grid_spec,
        compiler_params=pltpu.CompilerParams(
            dimension_semantics=("arbitrary",),
        ),
    )(tok, emb_table, w_ih_p, w_hh_p, b_rnn_p, w_fc_p, b_fc_p)

    return out_padded[:B, :V]


# speedup vs baseline: 1.1860x; 1.1860x over previous
import jax
import jax.numpy as jnp
from jax.experimental import pallas as pl
from jax.experimental.pallas import tpu as pltpu


def _round_up(x, m):
    return (x + m - 1) // m * m


def _rnn_core(
    tok_ref, emb_ref, wih_ref, whh_ref, brnn_ref, wfc_ref, bfc_ref, out_ref,
    xa_ref, xb_ref, pa_ref, pb_ref,
):
    bp = out_ref.shape[0]
    hp = whh_ref.shape[0]
    seq_len = tok_ref.shape[0] // bp

    w_ih = wih_ref[...].astype(jnp.bfloat16)
    w_hh = whh_ref[...].astype(jnp.bfloat16)
    b_rnn = brnn_ref[...]

    bufs = (xa_ref, xb_ref)

    def gather_to(buf, t):
        base = t * bp
        for i in range(bp):
            idx = tok_ref[base + i]
            buf[pl.ds(i, 1), :] = emb_ref[pl.ds(idx, 1), :]

    pbufs = (pa_ref, pb_ref)

    gather_to(bufs[0], 0)
    gather_to(bufs[1], 1)
    pa_ref[...] = (
        jnp.dot(
            bufs[0][...].astype(jnp.bfloat16),
            w_ih,
            preferred_element_type=jnp.float32,
        )
        + b_rnn
    )

    h = jnp.zeros((bp, hp), jnp.bfloat16)
    for t in range(seq_len):
        if t + 2 < seq_len:
            gather_to(bufs[t % 2], t + 2)
        if t + 1 < seq_len:
            pbufs[(t + 1) % 2][...] = (
                jnp.dot(
                    bufs[(t + 1) % 2][...].astype(jnp.bfloat16),
                    w_ih,
                    preferred_element_type=jnp.float32,
                )
                + b_rnn
            )
        acc = pbufs[t % 2][...] + jnp.dot(
            h, w_hh, preferred_element_type=jnp.float32
        )
        h = jnp.tanh(acc.astype(jnp.bfloat16))

    out_ref[...] = (
        jnp.dot(
            h.astype(jnp.float32),
            wfc_ref[...],
            preferred_element_type=jnp.float32,
        )
        + bfc_ref[...]
    )


def kernel(token_ids, emb_table, w_ih, w_hh, b_rnn, w_fc, b_fc):
    B, S = token_ids.shape
    E = emb_table.shape[1]
    H = w_ih.shape[1]
    V = w_fc.shape[1]

    Bp = _round_up(max(B, 8), 8)
    Hp = _round_up(H, 128)
    Vp = _round_up(V, 128)

    tok = jnp.pad(token_ids.T, ((0, 0), (0, Bp - B))).reshape(S * Bp)

    w_ih_p = jnp.pad(w_ih, ((0, 0), (0, Hp - H)))
    w_hh_p = jnp.pad(w_hh, ((0, Hp - H), (0, Hp - H)))
    b_rnn_p = jnp.pad(b_rnn, ((0, 0), (0, Hp - H)))
    w_fc_p = jnp.pad(w_fc, ((0, Hp - H), (0, Vp - V)))
    b_fc_p = jnp.pad(b_fc, ((0, 0), (0, Vp - V)))

    grid_spec = pltpu.PrefetchScalarGridSpec(
        num_scalar_prefetch=1,
        grid=(1,),
        in_specs=[
            pl.BlockSpec(emb_table.shape, lambda i, *_: (0, 0)),
            pl.BlockSpec((E, Hp), lambda i, *_: (0, 0)),
            pl.BlockSpec((Hp, Hp), lambda i, *_: (0, 0)),
            pl.BlockSpec((1, Hp), lambda i, *_: (0, 0)),
            pl.BlockSpec((Hp, Vp), lambda i, *_: (0, 0)),
            pl.BlockSpec((1, Vp), lambda i, *_: (0, 0)),
        ],
        out_specs=pl.BlockSpec((Bp, Vp), lambda i, *_: (0, 0)),
        scratch_shapes=[
            pltpu.VMEM((Bp, E), jnp.float32),
            pltpu.VMEM((Bp, E), jnp.float32),
            pltpu.VMEM((Bp, Hp), jnp.float32),
            pltpu.VMEM((Bp, Hp), jnp.float32),
        ],
    )

    out_padded = pl.pallas_call(
        _rnn_core,
        out_shape=jax.ShapeDtypeStruct((Bp, Vp), jnp.float32),
        grid_spec=grid_spec,
        compiler_params=pltpu.CompilerParams(
            dimension_semantics=("arbitrary",),
        ),
    )(tok, emb_table, w_ih_p, w_hh_p, b_rnn_p, w_fc_p, b_fc_p)

    return out_padded[:B, :V]
